# baseline (speedup 1.0000x reference)
import jax
import jax.numpy as jnp
from jax.experimental import pallas as pl
from jax.experimental.pallas import tpu as pltpu

_NEG = -100000.0


def _body(lat_ref, len_ref, pos_ref, pep_ref, wpos_ref, bpos_ref,
          wam_ref, bam_ref, out_pos_ref, out_am_ref, acc_ref):
    T = pl.num_programs(1)
    t = pl.program_id(1)
    x = lat_ref[0]
    w_row = wpos_ref[...]
    s = jax.lax.dot_general(w_row, x, (((1,), (1,)), ((), ())),
                            preferred_element_type=jnp.float32)
    s = s + bpos_ref[0, 0]
    lens_row = len_ref[...]
    out_pos_ref[...] = jnp.where(t < lens_row, s, _NEG).reshape(1, 1, -1)

    pos_ac = pos_ref[...]
    m = (pos_ac == t).astype(jnp.float32)
    contrib = m * x

    @pl.when(t == 0)
    def _():
        acc_ref[...] = contrib

    @pl.when(t > 0)
    def _():
        acc_ref[...] += contrib

    @pl.when(t == T - 1)
    def _():
        am = jax.lax.dot_general(acc_ref[...], wam_ref[...],
                                 (((1,), (1,)), ((), ())),
                                 preferred_element_type=jnp.float32)
        am = am + bam_ref[...]
        pep = pep_ref[...]
        BBB = pep.shape[0]
        lane_t = jax.lax.broadcasted_iota(jnp.int32, pep.shape, 1)
        pep_sel = jnp.sum(jnp.where(lane_t == pos_ac, pep, 0), axis=1,
                          keepdims=True)
        mask_col = jnp.where(pep_sel == 0, 19, pep_sel - 1)
        k_iota = jax.lax.broadcasted_iota(jnp.int32, (BBB, 20), 1)
        out_am_ref[...] = jnp.where(k_iota == mask_col, _NEG, am)


def kernel(latent_amino, latent_pep, peptides, alleles, lengths, pretrain,
           actions, W_pos, b_pos, W_amino, b_amino):
    T, B, D = latent_amino.shape
    BBB = 4096
    lengths2 = lengths.astype(jnp.int32).reshape(1, B)
    pos_ac = actions[:, 0:1].astype(jnp.int32)
    pep = peptides.astype(jnp.int32)
    bpos2 = b_pos.reshape(1, 1).astype(jnp.float32)
    bam2 = b_amino.reshape(1, -1).astype(jnp.float32)
    f = pl.pallas_call(
        _body,
        grid=(B // BBB, T),
        in_specs=[
            pl.BlockSpec((1, BBB, D), lambda c, t: (t, c, 0)),
            pl.BlockSpec((1, BBB), lambda c, t: (0, c)),
            pl.BlockSpec((BBB, 1), lambda c, t: (c, 0)),
            pl.BlockSpec((BBB, T), lambda c, t: (c, 0)),
            pl.BlockSpec((1, D), lambda c, t: (0, 0)),
            pl.BlockSpec((1, 1), lambda c, t: (0, 0)),
            pl.BlockSpec((20, D), lambda c, t: (0, 0)),
            pl.BlockSpec((1, 20), lambda c, t: (0, 0)),
        ],
        out_specs=(
            pl.BlockSpec((1, 1, BBB), lambda c, t: (t, 0, c)),
            pl.BlockSpec((BBB, 20), lambda c, t: (c, 0)),
        ),
        out_shape=(
            jax.ShapeDtypeStruct((T, 1, B), jnp.float32),
            jax.ShapeDtypeStruct((B, 20), jnp.float32),
        ),
        scratch_shapes=[pltpu.VMEM((BBB, D), jnp.float32)],
    )
    scores_T, amino_pd = f(latent_amino, lengths2, pos_ac, pep, W_pos,
                           bpos2, W_amino, bam2)
    return (scores_T.reshape(T, B).T, amino_pd)
